# baseline (speedup 1.0000x reference)
import functools

import jax
import jax.numpy as jnp
from jax import lax
from jax.experimental import pallas as pl
from jax.experimental.pallas import tpu as pltpu
from jax.experimental.pallas import tpu_sc as plsc

N = 10000
E = 320000
D = 128
H = 64
C = 40

NC = 2
NS = 16
NW = NC * NS

NP = 10240
NPT = NP // NS
CHN = 128
NB = 80
EPW = CHN * NB
EP = EPW * NW
EPR = EP // CHN
PD = 8
C48 = 48

_mesh = plsc.VectorSubcoreMesh(core_axis_name="c", subcore_axis_name="s")
_sc_params = pltpu.CompilerParams(use_tc_tiling_on_sc=False)


@functools.partial(
    pl.kernel,
    out_type=jax.ShapeDtypeStruct((NC, 2, NP), jnp.float32),
    mesh=_mesh,
    compiler_params=_sc_params,
    scratch_types=[
        pltpu.VMEM((8, CHN), jnp.int32),
        pltpu.VMEM((8, CHN), jnp.float32),
        pltpu.VMEM((CHN,), jnp.float32),
        pltpu.VMEM((NPT,), jnp.float32),
        pltpu.VMEM_SHARED((NP,), jnp.float32),
        pltpu.VMEM_SHARED((NP,), jnp.float32),
        pltpu.SemaphoreType.DMA,
    ],
)
def _sc_degrees(col_hbm, w_hbm, out_hbm, colb, wbb, ones, zb, degw, degc, sem):
    cid = lax.axis_index("c")
    sid = lax.axis_index("s")
    wid = cid * NS + sid

    ov = jnp.ones((16,), jnp.float32)
    zv = jnp.zeros((16,), jnp.float32)
    for i in range(CHN // 16):
        ones[pl.ds(i * 16, 16)] = ov

    @pl.loop(0, NPT // 16)
    def _(i):
        zb[pl.ds(i * 16, 16)] = zv

    my_rows = pl.ds(sid * NPT, NPT)
    pltpu.sync_copy(zb, degw.at[my_rows])
    pltpu.sync_copy(zb, degc.at[my_rows])
    plsc.subcore_barrier()

    @pl.loop(0, NB // 8)
    def _(t):
        base = pl.multiple_of(wid * NB + t * 8, 8)
        pltpu.sync_copy(col_hbm.at[pl.ds(base, 8)], colb)
        pltpu.sync_copy(w_hbm.at[pl.ds(base, 8)], wbb)
        descs = []
        for j in range(8):
            descs.append(pltpu.async_copy(wbb.at[j], degw.at[colb.at[j]], sem, add=True))
            descs.append(pltpu.async_copy(ones, degc.at[colb.at[j]], sem, add=True))
        for d in descs:
            d.wait()

    plsc.subcore_barrier()
    pltpu.sync_copy(degw.at[my_rows], out_hbm.at[cid, 0, my_rows])
    pltpu.sync_copy(degc.at[my_rows], out_hbm.at[cid, 1, my_rows])


HH = H // 2


@functools.partial(
    pl.kernel,
    out_type=jax.ShapeDtypeStruct((NC, 2, NP, HH), jnp.float32),
    mesh=_mesh,
    compiler_params=_sc_params,
    scratch_types=[
        pltpu.VMEM((2, PD, CHN), jnp.int32),
        pltpu.VMEM((2, PD, CHN), jnp.int32),
        pltpu.VMEM((2, PD, CHN), jnp.float32),
        pltpu.VMEM((PD, CHN, HH), jnp.float32),
        pltpu.VMEM((NPT // 4, HH), jnp.float32),
        pltpu.VMEM_SHARED((NP, HH), jnp.float32),
        pltpu.VMEM_SHARED((NP, HH), jnp.float32),
        pltpu.SemaphoreType.DMA,
        pltpu.SemaphoreType.DMA,
    ],
)
def _sc_agg1(row_hbm, col_hbm, w_hbm, x1h_hbm, out_hbm,
             rowb, colb, wbb, msg, zb, acc, xs, gsem, ssem):
    cid = lax.axis_index("c")
    sid = lax.axis_index("s")
    wid = cid * NS + sid

    zv = jnp.zeros((16,), jnp.float32)

    @pl.loop(0, NPT // 4)
    def _(r):
        for q in range(HH // 16):
            zb[r, pl.ds(q * 16, 16)] = zv

    my_rows = pl.ds(sid * NPT, NPT)

    def load_block(o, bb):
        base = pl.multiple_of(wid * NB + o * PD, PD)
        pltpu.sync_copy(row_hbm.at[pl.ds(base, PD)], rowb.at[bb])
        pltpu.sync_copy(col_hbm.at[pl.ds(base, PD)], colb.at[bb])
        pltpu.sync_copy(w_hbm.at[pl.ds(base, PD)], wbb.at[bb])

    for h in range(2):
        for kk in range(4):
            pltpu.sync_copy(zb, acc.at[pl.ds(sid * NPT + kk * (NPT // 4), NPT // 4)])
        pltpu.sync_copy(x1h_hbm.at[h, my_rows], xs.at[my_rows])
        plsc.subcore_barrier()

        load_block(0, 0)
        pltpu.async_copy(xs.at[rowb.at[0, 0]], msg.at[0], gsem)

        @pl.loop(0, NB // PD)
        def _(o):
            bb = o % 2
            for b in range(PD):
                i = o * PD + b
                nb = (b + 1) % PD

                @pl.when(i >= PD - 1)
                def _():
                    pltpu.make_async_copy(xs.at[pl.ds(0, CHN)], msg.at[nb], ssem).wait()

                if b == PD - 1:
                    @pl.when(i + 1 < NB)
                    def _():
                        load_block(o + 1, (o + 1) % 2)
                        pltpu.async_copy(xs.at[rowb.at[(o + 1) % 2, 0]], msg.at[0], gsem)
                else:
                    pltpu.async_copy(xs.at[rowb.at[bb, b + 1]], msg.at[nb], gsem)

                pltpu.make_async_copy(xs.at[pl.ds(0, CHN)], msg.at[b], gsem).wait()

                @pl.loop(0, CHN // 16)
                def _(g):
                    wv = wbb[bb, b, pl.ds(g * 16, 16)]
                    for j in range(16):
                        wvb = lax.broadcast_in_dim(wv[j], (16,), ())
                        for q in range(HH // 16):
                            sl = (b, g * 16 + j, pl.ds(q * 16, 16))
                            msg[sl] = msg[sl] * wvb

                pltpu.async_copy(msg.at[b], acc.at[colb.at[bb, b]], ssem, add=True)

        for k in range(PD - 1):
            b = (NB - (PD - 1) + k) % PD
            pltpu.make_async_copy(xs.at[pl.ds(0, CHN)], msg.at[b], ssem).wait()

        plsc.subcore_barrier()
        pltpu.sync_copy(acc.at[my_rows], out_hbm.at[cid, h, my_rows])


@functools.partial(
    pl.kernel,
    out_type=jax.ShapeDtypeStruct((NC, NP, C48), jnp.float32),
    mesh=_mesh,
    compiler_params=_sc_params,
    scratch_types=[
        pltpu.VMEM((2, PD, CHN), jnp.int32),
        pltpu.VMEM((2, PD, CHN), jnp.int32),
        pltpu.VMEM((PD, CHN, C48), jnp.float32),
        pltpu.VMEM((NPT // 4, C48), jnp.float32),
        pltpu.VMEM_SHARED((NP, C48), jnp.float32),
        pltpu.VMEM_SHARED((NP, C48), jnp.float32),
        pltpu.SemaphoreType.DMA,
        pltpu.SemaphoreType.DMA,
    ],
)
def _sc_agg2(row_hbm, col_hbm, t_hbm, out_hbm,
             rowb, colb, msg, zb, acc, ts, gsem, ssem):
    cid = lax.axis_index("c")
    sid = lax.axis_index("s")
    wid = cid * NS + sid

    zv = jnp.zeros((16,), jnp.float32)

    @pl.loop(0, NPT // 4)
    def _(r):
        for q in range(C48 // 16):
            zb[r, pl.ds(q * 16, 16)] = zv

    my_rows = pl.ds(sid * NPT, NPT)
    for kk in range(4):
        pltpu.sync_copy(zb, acc.at[pl.ds(sid * NPT + kk * (NPT // 4), NPT // 4)])
    pltpu.sync_copy(t_hbm.at[my_rows], ts.at[my_rows])
    plsc.subcore_barrier()

    def load_block(o, bb):
        base = pl.multiple_of(wid * NB + o * PD, PD)
        pltpu.sync_copy(row_hbm.at[pl.ds(base, PD)], rowb.at[bb])
        pltpu.sync_copy(col_hbm.at[pl.ds(base, PD)], colb.at[bb])

    load_block(0, 0)
    pltpu.async_copy(ts.at[rowb.at[0, 0]], msg.at[0], gsem)

    @pl.loop(0, NB // PD)
    def _(o):
        bb = o % 2
        for b in range(PD):
            i = o * PD + b
            nb = (b + 1) % PD

            @pl.when(i >= PD - 1)
            def _():
                pltpu.make_async_copy(t_hbm.at[pl.ds(0, CHN)], msg.at[nb], ssem).wait()

            if b == PD - 1:
                @pl.when(i + 1 < NB)
                def _():
                    load_block(o + 1, (o + 1) % 2)
                    pltpu.async_copy(ts.at[rowb.at[(o + 1) % 2, 0]], msg.at[0], gsem)
            else:
                pltpu.async_copy(ts.at[rowb.at[bb, b + 1]], msg.at[nb], gsem)

            pltpu.make_async_copy(t_hbm.at[pl.ds(0, CHN)], msg.at[b], gsem).wait()
            pltpu.async_copy(msg.at[b], acc.at[colb.at[bb, b]], ssem, add=True)

    for k in range(PD - 1):
        b = (NB - (PD - 1) + k) % PD
        pltpu.make_async_copy(t_hbm.at[pl.ds(0, CHN)], msg.at[b], ssem).wait()

    plsc.subcore_barrier()
    pltpu.sync_copy(acc.at[my_rows], out_hbm.at[cid, my_rows])


def _tc_x1_body(x_ref, w1_ref, deg1_ref, deg2_ref, x1_ref, x1h_ref,
                d1_ref, d2_ref):
    d1 = lax.rsqrt(deg1_ref[...])
    d2 = lax.rsqrt(deg2_ref[...])
    xw = jnp.dot(x_ref[...], w1_ref[...], preferred_element_type=jnp.float32)
    x1 = xw * d1
    x1_ref[...] = x1
    x1h_ref[0] = x1[:, :HH]
    x1h_ref[1] = x1[:, HH:]
    d1_ref[...] = d1
    d2_ref[...] = d2


_tc_x1 = pl.pallas_call(
    _tc_x1_body,
    out_shape=[
        jax.ShapeDtypeStruct((NP, H), jnp.float32),
        jax.ShapeDtypeStruct((2, NP, HH), jnp.float32),
        jax.ShapeDtypeStruct((NP, 1), jnp.float32),
        jax.ShapeDtypeStruct((NP, 1), jnp.float32),
    ],
)


def _tc_mid_body(accp_ref, x1_ref, d1_ref, d2_ref, b1_ref, w2_ref, t_ref):
    accs = jnp.concatenate(
        [accp_ref[0, 0] + accp_ref[1, 0], accp_ref[0, 1] + accp_ref[1, 1]],
        axis=1)
    a = accs + x1_ref[...]
    h = jnp.maximum(d1_ref[...] * a + b1_ref[...], 0.0)
    t_ref[...] = jnp.dot(h, w2_ref[...], preferred_element_type=jnp.float32) * d2_ref[...]


_tc_mid = pl.pallas_call(
    _tc_mid_body,
    out_shape=jax.ShapeDtypeStruct((NP, C48), jnp.float32),
)


def _tc_out_body(accp_ref, t_ref, d2_ref, b2_ref, o_ref):
    lg = d2_ref[...] * (accp_ref[0] + accp_ref[1] + t_ref[...]) + b2_ref[...]
    mask = lax.broadcasted_iota(jnp.int32, (NP, C48), 1) < C
    l = jnp.where(mask, lg, -1e30)
    mx = jnp.max(l, axis=1, keepdims=True)
    s = jnp.sum(jnp.exp(l - mx), axis=1, keepdims=True)
    o_ref[...] = l - mx - jnp.log(s)


_tc_out = pl.pallas_call(
    _tc_out_body,
    out_shape=jax.ShapeDtypeStruct((NP, C48), jnp.float32),
)


def kernel(x, edge_index, edge_weight, W1, b1, W2, b2):
    f32 = jnp.float32
    row = edge_index[0]
    col = edge_index[1]

    npad = EP - E
    pad_idx = jnp.arange(npad, dtype=jnp.int32)
    pad_dst = (N + pad_idx % (NP - N)).astype(jnp.int32)
    rowp = jnp.concatenate([row, pad_idx % N])
    colp = jnp.concatenate([col, pad_dst])
    wp = jnp.concatenate([edge_weight.astype(f32), jnp.zeros((npad,), f32)])

    xp = jnp.zeros((NP, D), f32).at[:N].set(x.astype(f32))
    w2p = jnp.zeros((H, C48), f32).at[:, :C].set(W2.astype(f32))
    b2p = jnp.zeros((1, C48), f32).at[0, :C].set(b2.astype(f32))
    b1r = b1.astype(f32).reshape(1, H)

    degs = _sc_degrees(colp.reshape(EPR, CHN), wp.reshape(EPR, CHN))
    deg1 = (degs[0, 0] + degs[1, 0] + 1.0)[:, None]
    deg2 = (degs[0, 1] + degs[1, 1] + 1.0)[:, None]

    x1, x1h, d1, d2 = _tc_x1(xp, W1.astype(f32), deg1, deg2)
    rp2 = rowp.reshape(EPR, CHN)
    cp2 = colp.reshape(EPR, CHN)
    wp2 = wp.reshape(EPR, CHN)
    acc1 = _sc_agg1(rp2, cp2, wp2, x1h)
    t = _tc_mid(acc1, x1, d1, d2, b1r, w2p)
    acc2 = _sc_agg2(rp2, cp2, t)
    o = _tc_out(acc2, t, d2, b2p)
    return o[:N, :C]


# speedup vs baseline: 1.0330x; 1.0330x over previous
import functools

import jax
import jax.numpy as jnp
from jax import lax
from jax.experimental import pallas as pl
from jax.experimental.pallas import tpu as pltpu
from jax.experimental.pallas import tpu_sc as plsc

N = 10000
E = 320000
D = 128
H = 64
C = 40

NC = 2
NS = 16
NW = NC * NS

NP = 10240
NPT = NP // NS
CHN = 128
NB = 80
EPW = CHN * NB
EP = EPW * NW
EPR = EP // CHN
PD = 8
C48 = 48

_mesh = plsc.VectorSubcoreMesh(core_axis_name="c", subcore_axis_name="s")
_sc_params = pltpu.CompilerParams(use_tc_tiling_on_sc=False)


@functools.partial(
    pl.kernel,
    out_type=jax.ShapeDtypeStruct((NC, 2, NP), jnp.float32),
    mesh=_mesh,
    compiler_params=_sc_params,
    scratch_types=[
        pltpu.VMEM((8, CHN), jnp.int32),
        pltpu.VMEM((8, CHN), jnp.float32),
        pltpu.VMEM((CHN,), jnp.float32),
        pltpu.VMEM((NPT,), jnp.float32),
        pltpu.VMEM_SHARED((NP,), jnp.float32),
        pltpu.VMEM_SHARED((NP,), jnp.float32),
        pltpu.SemaphoreType.DMA,
    ],
)
def _sc_degrees(col_hbm, w_hbm, out_hbm, colb, wbb, ones, zb, degw, degc, sem):
    cid = lax.axis_index("c")
    sid = lax.axis_index("s")
    wid = cid * NS + sid

    ov = jnp.ones((16,), jnp.float32)
    zv = jnp.zeros((16,), jnp.float32)
    for i in range(CHN // 16):
        ones[pl.ds(i * 16, 16)] = ov

    @pl.loop(0, NPT // 16)
    def _(i):
        zb[pl.ds(i * 16, 16)] = zv

    my_rows = pl.ds(sid * NPT, NPT)
    pltpu.sync_copy(zb, degw.at[my_rows])
    pltpu.sync_copy(zb, degc.at[my_rows])
    plsc.subcore_barrier()

    @pl.loop(0, NB // 8)
    def _(t):
        base = pl.multiple_of(wid * NB + t * 8, 8)
        pltpu.sync_copy(col_hbm.at[pl.ds(base, 8)], colb)
        pltpu.sync_copy(w_hbm.at[pl.ds(base, 8)], wbb)
        descs = []
        for j in range(8):
            descs.append(pltpu.async_copy(wbb.at[j], degw.at[colb.at[j]], sem, add=True))
            descs.append(pltpu.async_copy(ones, degc.at[colb.at[j]], sem, add=True))
        for d in descs:
            d.wait()

    plsc.subcore_barrier()
    pltpu.sync_copy(degw.at[my_rows], out_hbm.at[cid, 0, my_rows])
    pltpu.sync_copy(degc.at[my_rows], out_hbm.at[cid, 1, my_rows])


@functools.partial(
    pl.kernel,
    out_type=jax.ShapeDtypeStruct((NC, NP, H), jnp.float32),
    mesh=_mesh,
    compiler_params=_sc_params,
    scratch_types=[
        pltpu.VMEM((2, PD, CHN), jnp.int32),
        pltpu.VMEM((2, PD, CHN), jnp.int32),
        pltpu.VMEM((2, PD, CHN), jnp.float32),
        pltpu.VMEM((PD, CHN, H), jnp.float32),
        pltpu.VMEM((NPT // 4, H), jnp.float32),
        pltpu.VMEM_SHARED((NP, H), jnp.float32),
        pltpu.SemaphoreType.DMA,
        pltpu.SemaphoreType.DMA,
    ],
)
def _sc_agg1(row_hbm, col_hbm, w_hbm, x1_hbm, out_hbm,
             rowb, colb, wbb, msg, zb, acc, gsem, ssem):
    cid = lax.axis_index("c")
    sid = lax.axis_index("s")
    wid = cid * NS + sid

    zv = jnp.zeros((16,), jnp.float32)

    @pl.loop(0, NPT // 4)
    def _(r):
        for q in range(H // 16):
            zb[r, pl.ds(q * 16, 16)] = zv

    my_rows = pl.ds(sid * NPT, NPT)
    for kk in range(4):
        pltpu.sync_copy(zb, acc.at[pl.ds(sid * NPT + kk * (NPT // 4), NPT // 4)])
    plsc.subcore_barrier()

    def load_block(o, bb):
        base = pl.multiple_of(wid * NB + o * PD, PD)
        pltpu.sync_copy(row_hbm.at[pl.ds(base, PD)], rowb.at[bb])
        pltpu.sync_copy(col_hbm.at[pl.ds(base, PD)], colb.at[bb])
        pltpu.sync_copy(w_hbm.at[pl.ds(base, PD)], wbb.at[bb])

    load_block(0, 0)
    pltpu.async_copy(x1_hbm.at[rowb.at[0, 0]], msg.at[0], gsem)

    @pl.loop(0, NB // PD)
    def _(o):
        bb = o % 2
        for b in range(PD):
            i = o * PD + b
            nb = (b + 1) % PD

            @pl.when(i >= PD - 1)
            def _():
                pltpu.make_async_copy(x1_hbm.at[pl.ds(0, CHN)], msg.at[nb], ssem).wait()

            if b == PD - 1:
                @pl.when(i + 1 < NB)
                def _():
                    load_block(o + 1, (o + 1) % 2)
                    pltpu.async_copy(x1_hbm.at[rowb.at[(o + 1) % 2, 0]], msg.at[0], gsem)
            else:
                pltpu.async_copy(x1_hbm.at[rowb.at[bb, b + 1]], msg.at[nb], gsem)

            pltpu.make_async_copy(x1_hbm.at[pl.ds(0, CHN)], msg.at[b], gsem).wait()

            @pl.loop(0, CHN // 16)
            def _(g):
                wv = wbb[bb, b, pl.ds(g * 16, 16)]
                for j in range(16):
                    wvb = lax.broadcast_in_dim(wv[j], (16,), ())
                    for q in range(H // 16):
                        sl = (b, g * 16 + j, pl.ds(q * 16, 16))
                        msg[sl] = msg[sl] * wvb

            pltpu.async_copy(msg.at[b], acc.at[colb.at[bb, b]], ssem, add=True)

    for k in range(PD - 1):
        b = (NB - (PD - 1) + k) % PD
        pltpu.make_async_copy(x1_hbm.at[pl.ds(0, CHN)], msg.at[b], ssem).wait()

    plsc.subcore_barrier()
    pltpu.sync_copy(acc.at[my_rows], out_hbm.at[cid, my_rows])


@functools.partial(
    pl.kernel,
    out_type=jax.ShapeDtypeStruct((NC, NP, C48), jnp.float32),
    mesh=_mesh,
    compiler_params=_sc_params,
    scratch_types=[
        pltpu.VMEM((2, PD, CHN), jnp.int32),
        pltpu.VMEM((2, PD, CHN), jnp.int32),
        pltpu.VMEM((PD, CHN, C48), jnp.float32),
        pltpu.VMEM((NPT // 4, C48), jnp.float32),
        pltpu.VMEM_SHARED((NP, C48), jnp.float32),
        pltpu.VMEM_SHARED((NP, C48), jnp.float32),
        pltpu.SemaphoreType.DMA,
        pltpu.SemaphoreType.DMA,
    ],
)
def _sc_agg2(row_hbm, col_hbm, t_hbm, out_hbm,
             rowb, colb, msg, zb, acc, ts, gsem, ssem):
    cid = lax.axis_index("c")
    sid = lax.axis_index("s")
    wid = cid * NS + sid

    zv = jnp.zeros((16,), jnp.float32)

    @pl.loop(0, NPT // 4)
    def _(r):
        for q in range(C48 // 16):
            zb[r, pl.ds(q * 16, 16)] = zv

    my_rows = pl.ds(sid * NPT, NPT)
    for kk in range(4):
        pltpu.sync_copy(zb, acc.at[pl.ds(sid * NPT + kk * (NPT // 4), NPT // 4)])
    pltpu.sync_copy(t_hbm.at[my_rows], ts.at[my_rows])
    plsc.subcore_barrier()

    def load_block(o, bb):
        base = pl.multiple_of(wid * NB + o * PD, PD)
        pltpu.sync_copy(row_hbm.at[pl.ds(base, PD)], rowb.at[bb])
        pltpu.sync_copy(col_hbm.at[pl.ds(base, PD)], colb.at[bb])

    load_block(0, 0)
    pltpu.async_copy(ts.at[rowb.at[0, 0]], msg.at[0], gsem)

    @pl.loop(0, NB // PD)
    def _(o):
        bb = o % 2
        for b in range(PD):
            i = o * PD + b
            nb = (b + 1) % PD

            @pl.when(i >= PD - 1)
            def _():
                pltpu.make_async_copy(t_hbm.at[pl.ds(0, CHN)], msg.at[nb], ssem).wait()

            if b == PD - 1:
                @pl.when(i + 1 < NB)
                def _():
                    load_block(o + 1, (o + 1) % 2)
                    pltpu.async_copy(ts.at[rowb.at[(o + 1) % 2, 0]], msg.at[0], gsem)
            else:
                pltpu.async_copy(ts.at[rowb.at[bb, b + 1]], msg.at[nb], gsem)

            pltpu.make_async_copy(t_hbm.at[pl.ds(0, CHN)], msg.at[b], gsem).wait()
            pltpu.async_copy(msg.at[b], acc.at[colb.at[bb, b]], ssem, add=True)

    for k in range(PD - 1):
        b = (NB - (PD - 1) + k) % PD
        pltpu.make_async_copy(t_hbm.at[pl.ds(0, CHN)], msg.at[b], ssem).wait()

    plsc.subcore_barrier()
    pltpu.sync_copy(acc.at[my_rows], out_hbm.at[cid, my_rows])


def _tc_x1_body(x_ref, w1_ref, deg1_ref, deg2_ref, x1_ref, d1_ref, d2_ref):
    d1 = lax.rsqrt(deg1_ref[...])
    d2 = lax.rsqrt(deg2_ref[...])
    xw = jnp.dot(x_ref[...], w1_ref[...], preferred_element_type=jnp.float32)
    x1_ref[pl.ds(0, N)] = xw * d1[:N]
    x1_ref[pl.ds(N, NP - N)] = jnp.zeros((NP - N, H), jnp.float32)
    d1_ref[...] = d1
    d2_ref[...] = d2


_tc_x1 = pl.pallas_call(
    _tc_x1_body,
    out_shape=[
        jax.ShapeDtypeStruct((NP, H), jnp.float32),
        jax.ShapeDtypeStruct((NP, 1), jnp.float32),
        jax.ShapeDtypeStruct((NP, 1), jnp.float32),
    ],
)


def _tc_mid_body(accp_ref, x1_ref, d1_ref, d2_ref, b1_ref, w2_ref, t_ref):
    a = accp_ref[0] + accp_ref[1] + x1_ref[...]
    h = jnp.maximum(d1_ref[...] * a + b1_ref[...], 0.0)
    t_ref[...] = jnp.dot(h, w2_ref[...], preferred_element_type=jnp.float32) * d2_ref[...]


_tc_mid = pl.pallas_call(
    _tc_mid_body,
    out_shape=jax.ShapeDtypeStruct((NP, C48), jnp.float32),
)


def _tc_out_body(accp_ref, t_ref, d2_ref, b2_ref, o_ref):
    lg = d2_ref[...] * (accp_ref[0] + accp_ref[1] + t_ref[...]) + b2_ref[...]
    mask = lax.broadcasted_iota(jnp.int32, (NP, C48), 1) < C
    l = jnp.where(mask, lg, -1e30)
    mx = jnp.max(l, axis=1, keepdims=True)
    s = jnp.sum(jnp.exp(l - mx), axis=1, keepdims=True)
    o_ref[...] = l - mx - jnp.log(s)


_tc_out = pl.pallas_call(
    _tc_out_body,
    out_shape=jax.ShapeDtypeStruct((NP, C48), jnp.float32),
)


def kernel(x, edge_index, edge_weight, W1, b1, W2, b2):
    f32 = jnp.float32
    row = edge_index[0]
    col = edge_index[1]

    npad = EP - E
    pad_idx = jnp.arange(npad, dtype=jnp.int32)
    pad_dst = (N + pad_idx % (NP - N)).astype(jnp.int32)
    rowp = jnp.concatenate([row, pad_idx % N])
    colp = jnp.concatenate([col, pad_dst])
    wp = jnp.concatenate([edge_weight.astype(f32), jnp.zeros((npad,), f32)])

    w2p = jnp.zeros((H, C48), f32).at[:, :C].set(W2.astype(f32))
    b2p = jnp.zeros((1, C48), f32).at[0, :C].set(b2.astype(f32))
    b1r = b1.astype(f32).reshape(1, H)

    degs = _sc_degrees(colp.reshape(EPR, CHN), wp.reshape(EPR, CHN))
    deg1 = (degs[0, 0] + degs[1, 0] + 1.0)[:, None]
    deg2 = (degs[0, 1] + degs[1, 1] + 1.0)[:, None]

    x1, d1, d2 = _tc_x1(x.astype(f32), W1.astype(f32), deg1, deg2)
    rp2 = rowp.reshape(EPR, CHN)
    cp2 = colp.reshape(EPR, CHN)
    wp2 = wp.reshape(EPR, CHN)
    acc1 = _sc_agg1(rp2, cp2, wp2, x1)
    t = _tc_mid(acc1, x1, d1, d2, b1r, w2p)
    acc2 = _sc_agg2(rp2, cp2, t)
    o = _tc_out(acc2, t, d2, b2p)
    return o[:N, :C]
